# baseline (speedup 1.0000x reference)
import functools

import jax
import jax.numpy as jnp
from jax.experimental import pallas as pl
from jax.experimental.pallas import tpu as pltpu

N = 10000
E = 160000
F_IN = 256
HID = 256
HEADS = 8
CLASSES = 10
NGRAPH = 16
EPS_BN = 1e-5

EI = E + N


def _stats_kernel(x_ref, out_ref):
    i = pl.program_id(0)

    @pl.when(i == 0)
    def _():
        out_ref[...] = jnp.zeros_like(out_ref)

    xb = x_ref[...]
    s = jnp.sum(xb, axis=0)
    ss = jnp.sum(xb * xb, axis=0)
    out_ref[0, :] += s
    out_ref[1, :] += ss


def _col_stats(x, block_rows):
    n, f = x.shape
    grid = n // block_rows
    return pl.pallas_call(
        _stats_kernel,
        grid=(grid,),
        in_specs=[pl.BlockSpec((block_rows, f), lambda i: (i, 0))],
        out_specs=pl.BlockSpec((2, f), lambda i: (0, 0)),
        out_shape=jax.ShapeDtypeStruct((2, f), jnp.float32),
    )(x)


def _mm_att_kernel(x_ref, w_ref, scale_ref, shift_ref, vsrc_ref, vdst_ref,
                   h_ref, asrc_ref, adst_ref):
    xb = x_ref[...] * scale_ref[0, :] + shift_ref[0, :]
    h_ref[...] = jnp.dot(xb, w_ref[...], preferred_element_type=jnp.float32)
    asrc_ref[...] = jnp.dot(xb, vsrc_ref[...], preferred_element_type=jnp.float32)
    adst_ref[...] = jnp.dot(xb, vdst_ref[...], preferred_element_type=jnp.float32)


def _bn_matmul_att(x, w, scale, shift, v_src, v_dst, block_rows):
    n, f = x.shape
    fout = w.shape[1]
    heads = v_src.shape[1]
    grid = n // block_rows
    return pl.pallas_call(
        _mm_att_kernel,
        grid=(grid,),
        in_specs=[
            pl.BlockSpec((block_rows, f), lambda i: (i, 0)),
            pl.BlockSpec((f, fout), lambda i: (0, 0)),
            pl.BlockSpec((1, f), lambda i: (0, 0)),
            pl.BlockSpec((1, f), lambda i: (0, 0)),
            pl.BlockSpec((f, heads), lambda i: (0, 0)),
            pl.BlockSpec((f, heads), lambda i: (0, 0)),
        ],
        out_specs=[
            pl.BlockSpec((block_rows, fout), lambda i: (i, 0)),
            pl.BlockSpec((block_rows, heads), lambda i: (i, 0)),
            pl.BlockSpec((block_rows, heads), lambda i: (i, 0)),
        ],
        out_shape=[
            jax.ShapeDtypeStruct((n, fout), jnp.float32),
            jax.ShapeDtypeStruct((n, heads), jnp.float32),
            jax.ShapeDtypeStruct((n, heads), jnp.float32),
        ],
    )(x, w, scale, shift, v_src, v_dst)


def _edge_softmax_aggregate(h, a_src, a_dst, src, dst, heads, out_ch):
    n = h.shape[0]
    alpha = a_src[src] + a_dst[dst]
    alpha = jax.nn.leaky_relu(alpha, negative_slope=0.2)
    amax = jax.ops.segment_max(alpha, dst, num_segments=n)
    amax = jnp.where(jnp.isfinite(amax), amax, 0.0)
    ex = jnp.exp(alpha - amax[dst])
    denom = jax.ops.segment_sum(ex, dst, num_segments=n)
    alpha = ex / (denom[dst] + 1e-16)
    msg = h.reshape(n, heads, out_ch)[src] * alpha[:, :, None]
    out = jax.ops.segment_sum(msg, dst, num_segments=n)
    return out, alpha


def kernel(x, edge_index, batch, gamma1, beta1, W1, att_src1, att_dst1, bias1,
           gamma2, beta2, W2, att_src2, att_dst2, bias2):
    n = x.shape[0]
    sl = jnp.arange(n, dtype=edge_index.dtype)
    ei = jnp.concatenate([edge_index, jnp.stack([sl, sl])], axis=1)
    src, dst = ei[0], ei[1]

    stats = _col_stats(x, 1000)
    mean = stats[0] / n
    var = stats[1] / n - mean * mean
    scale = gamma1 / jnp.sqrt(var + EPS_BN)
    shift = beta1 - mean * scale
    w1r = W1.reshape(F_IN, HEADS, HID)
    v_src1 = jnp.einsum("chk,hk->ch", w1r, att_src1)
    v_dst1 = jnp.einsum("chk,hk->ch", w1r, att_dst1)
    h1, as1, ad1 = _bn_matmul_att(x, W1, scale[None, :], shift[None, :],
                                  v_src1, v_dst1, 1000)

    out1, alpha1 = _edge_softmax_aggregate(h1, as1, ad1, src, dst, HEADS, HID)
    out1 = out1.reshape(n, HEADS * HID) + bias1

    r = jax.nn.relu(out1)
    stats2 = _col_stats(r, 1000)
    mean2 = stats2[0] / n
    var2 = stats2[1] / n - mean2 * mean2
    scale2 = gamma2 / jnp.sqrt(var2 + EPS_BN)
    shift2 = beta2 - mean2 * scale2
    v_src2 = W2 @ att_src2[0]
    v_dst2 = W2 @ att_dst2[0]
    h2, as2, ad2 = _bn_matmul_att(r, W2, scale2[None, :], shift2[None, :],
                                  v_src2[:, None], v_dst2[:, None], 1000)

    out2, alpha2 = _edge_softmax_aggregate(h2, as2, ad2, src, dst, 1, CLASSES)
    out2 = out2[:, 0, :] + bias2

    hf = jax.nn.elu(out2)
    sums = jax.ops.segment_sum(hf, batch, num_segments=NGRAPH)
    counts = jax.ops.segment_sum(jnp.ones((n,), hf.dtype), batch,
                                 num_segments=NGRAPH)
    pooled = sums / jnp.maximum(counts, 1.0)[:, None]
    logp = jax.nn.log_softmax(pooled, axis=1)
    return (logp, (ei, alpha1), (ei, alpha2))
